# Initial kernel scaffold; baseline (speedup 1.0000x reference)
import jax
import jax.numpy as jnp
from jax.experimental import pallas as pl


def kernel(inp_0, edge_index_0, inp_1, edge_index_1, params):
    raise NotImplementedError("write your pallas kernel here")


# speedup vs baseline: 2.0379x; 2.0379x over previous
import functools

import jax
import jax.numpy as jnp
from jax.experimental import pallas as pl

_N = 32000
_DIMS = [(128, 16), (16, 8), (8, 4)] + [(4, 4)] * 20 + [(4, 8), (8, 16), (16, 128)]
_ENS = 2


def _scale_bias_relu_kernel(acc_ref, dinv_ref, b_ref, o_ref, *, relu):
    v = acc_ref[...] * dinv_ref[...] + b_ref[...]
    if relu:
        v = jnp.maximum(v, 0.0)
    o_ref[...] = v


def _scale_bias_relu(acc, dinv, b, relu):
    n, w = acc.shape
    bn = 8000
    return pl.pallas_call(
        functools.partial(_scale_bias_relu_kernel, relu=relu),
        grid=(n // bn,),
        in_specs=[
            pl.BlockSpec((bn, w), lambda i: (i, 0)),
            pl.BlockSpec((bn, 1), lambda i: (i, 0)),
            pl.BlockSpec((1, w), lambda i: (0, 0)),
        ],
        out_specs=pl.BlockSpec((bn, w), lambda i: (i, 0)),
        out_shape=jax.ShapeDtypeStruct((n, w), jnp.float32),
    )(acc, dinv.reshape(n, 1), b.reshape(1, w))


def _member(x, ei, Ws, bs):
    loop = jnp.arange(_N, dtype=ei.dtype)
    src = jnp.concatenate([ei[0], loop])
    dst = jnp.concatenate([ei[1], loop])
    deg = jax.ops.segment_sum(jnp.ones(src.shape[0], jnp.float32), dst,
                              num_segments=_N)
    dinv = jax.lax.rsqrt(deg)

    def spmv(t):
        msg = (t * dinv[:, None])[src]
        return jax.ops.segment_sum(msg, dst, num_segments=_N)

    h = x @ Ws[0]
    for j in range(0, 23):
        if j > 0:
            h = h @ Ws[j]
        acc = spmv(h)
        h = _scale_bias_relu(acc, dinv, bs[j], relu=True)
    for j in range(23, 26):
        acc = spmv(h)
        u = _scale_bias_relu(acc, dinv, jnp.zeros_like(bs[j][:1]).repeat(acc.shape[1]), relu=False)
        h = u @ Ws[j] + bs[j]
        if j < 25:
            h = jnp.maximum(h, 0.0)
    return h


def kernel(inp_0, edge_index_0, inp_1, edge_index_1, params):
    outs = []
    for i, (x, ei) in enumerate(((inp_0, edge_index_0), (inp_1, edge_index_1))):
        Ws = [params[f"W_{i}_{j}"] for j in range(1, 27)]
        bs = [params[f"b_{i}_{j}"] for j in range(1, 27)]
        outs.append(_member(x, ei, Ws, bs))
    return tuple(outs)


# speedup vs baseline: 38.2209x; 18.7550x over previous
import functools

import jax
import jax.numpy as jnp
from jax import lax
from jax.experimental import pallas as pl
from jax.experimental.pallas import tpu as pltpu
from jax.experimental.pallas import tpu_sc as plsc

N = 32000
E = 512000
NS = 16
NODES_T = N // NS
EDGES_T = E // NS
BLK = 128
KF = 10
NSB = EDGES_T // (BLK * KF)
CHUNK = 400
NCHUNK = NODES_T // CHUNK
NGRP = CHUNK // 16

WV_LEN = 10496
BV_LEN = 2112
OFF_D1, OFF_D2, OFF_DLOOP, OFF_D24, OFF_D25 = 0, 2048, 2560, 7936, 8448
BOFF_D1, BOFF_D2, BOFF_DLOOP, BOFF_D24, BOFF_D25 = 0, 256, 384, 1728, 1856


def _sc_body(y1h, srcah, dsth, wvh, bvh, u26h, tbl8h, tbl16h,
             acc8, acc16, deg,
             sidxb, didxb, buf8, buf16, ones2, dinv_t, wvs, bvs,
             gsem, ssem):
    cid = lax.axis_index("c")
    tid = lax.axis_index("s")
    iota = lax.iota(jnp.int32, 16)

    def edge_pass(tbl, acc, buf):
        def sb_body(sb, carry):
            pltpu.sync_copy(srcah.at[cid, tid, sb], sidxb)
            pltpu.sync_copy(dsth.at[cid, tid, sb], didxb)
            descs = []
            for j in range(KF):
                descs.append(pltpu.async_copy(
                    tbl.at[sidxb.at[j]], buf.at[pl.ds(j * BLK, BLK), :],
                    gsem))
            for d in descs:
                d.wait()
            descs = []
            for j in range(KF):
                descs.append(pltpu.async_copy(
                    buf.at[pl.ds(j * BLK, BLK), :], acc.at[didxb.at[j]],
                    ssem, add=True))
            for d in descs:
                d.wait()
            return carry
        lax.fori_loop(0, NSB, sb_body, 0)

    def load_wb(wv_off, wv_len, bv_off, bv_len):
        pltpu.sync_copy(
            wvh.at[pl.ds(pl.multiple_of(cid * WV_LEN + wv_off, 16), wv_len)],
            wvs.at[pl.ds(0, wv_len)])
        pltpu.sync_copy(
            bvh.at[pl.ds(pl.multiple_of(cid * BV_LEN + bv_off, 16), bv_len)],
            bvs.at[pl.ds(0, bv_len)])

    def dense_stage(src_ref, w_in, buf_in, w_out, buf_out, outs, mode,
                    src_is_hbm=False, w_pad=0):
        def chunk_body(c, carry):
            nbase = pl.multiple_of(tid * NODES_T + c * CHUNK, 16)
            if src_is_hbm:
                pltpu.sync_copy(src_ref.at[cid, pl.ds(nbase, CHUNK), :],
                                buf_in.at[pl.ds(0, CHUNK), :])
            else:
                pltpu.sync_copy(src_ref.at[pl.ds(nbase, CHUNK), :],
                                buf_in.at[pl.ds(0, CHUNK), :])

            def grp(j, carry2):
                rows = j * 16 + iota
                dv = dinv_t[pl.ds(pl.multiple_of(c * CHUNK + j * 16, 16), 16)]
                xs = []
                for ci in range(w_in):
                    v = plsc.load_gather(
                        buf_in, [rows, jnp.full((16,), ci, jnp.int32)])
                    v = v * dv
                    if mode == "pre":
                        b = bvs[pl.ds(ci * 16, 16)]
                        v = jnp.maximum(v + b, 0.0)
                    xs.append(v)
                if mode == "copy":
                    for co in range(w_out):
                        plsc.store_scatter(
                            buf_out, [rows, jnp.full((16,), co, jnp.int32)],
                            xs[co])
                else:
                    for co in range(w_out):
                        accv = None
                        for ci in range(w_in):
                            wvec = wvs[pl.ds((ci * w_out + co) * 16, 16)]
                            t = xs[ci] * wvec
                            accv = t if accv is None else accv + t
                        if mode == "post":
                            b = bvs[pl.ds(co * 16, 16)]
                            accv = jnp.maximum(accv + b, 0.0)
                        accv = accv * dv
                        plsc.store_scatter(
                            buf_out, [rows, jnp.full((16,), co, jnp.int32)],
                            accv)
                    for co in range(w_out, w_pad):
                        plsc.store_scatter(
                            buf_out, [rows, jnp.full((16,), co, jnp.int32)],
                            jnp.full((16,), 0.0, jnp.float32))
                return carry2
            lax.fori_loop(0, NGRP, grp, 0)

            src_chunk = buf_out.at[pl.ds(0, CHUNK), :]
            for kind, ref in outs:
                if kind == "sp":
                    pltpu.sync_copy(src_chunk, ref.at[pl.ds(nbase, CHUNK), :])
                elif kind == "hbm2":
                    pltpu.sync_copy(src_chunk,
                                    ref.at[cid, pl.ds(nbase, CHUNK), :])
                else:
                    pltpu.sync_copy(
                        src_chunk,
                        ref.at[pl.ds(pl.multiple_of(cid * N + nbase, 16), CHUNK), :])
            return carry
        lax.fori_loop(0, NCHUNK, chunk_body, 0)

    def fill_ones(g, carry):
        f = g * 16 + iota
        plsc.store_scatter(ones2, [f >> 3, f & 7],
                           jnp.full((16,), 1.0, jnp.float32))
        return carry
    lax.fori_loop(0, 400 * 8 // 16, fill_ones, 0)

    def deg_init(c, carry):
        pltpu.sync_copy(
            ones2,
            deg.at[pl.ds(pl.multiple_of(tid * NODES_T + c * CHUNK, 16),
                         CHUNK), :])
        return carry
    lax.fori_loop(0, NCHUNK, deg_init, 0)
    plsc.subcore_barrier()

    def deg_sb(sb, carry):
        pltpu.sync_copy(dsth.at[cid, tid, sb], didxb)
        descs = []
        for j in range(KF):
            descs.append(pltpu.async_copy(
                ones2.at[pl.ds(0, BLK), :], deg.at[didxb.at[j]],
                ssem, add=True))
        for d in descs:
            d.wait()
        return carry
    lax.fori_loop(0, NSB, deg_sb, 0)
    plsc.subcore_barrier()

    def newton_chunk(c, carry):
        pltpu.sync_copy(
            deg.at[pl.ds(pl.multiple_of(tid * NODES_T + c * CHUNK, 16),
                         CHUNK), :],
            buf8.at[pl.ds(0, CHUNK), :])

        def newton(j, carry2):
            rows = j * 16 + iota
            v = plsc.load_gather(buf8, [rows, jnp.full((16,), 0, jnp.int32)])
            bits = lax.bitcast_convert_type(v, jnp.int32)
            y = lax.bitcast_convert_type(
                jnp.int32(0x5F3759DF) - (bits >> 1), jnp.float32)
            for _ in range(3):
                y = y * (1.5 - 0.5 * v * y * y)
            dinv_t[pl.ds(pl.multiple_of(c * CHUNK + j * 16, 16), 16)] = y
            return carry2
        lax.fori_loop(0, NGRP, newton, 0)
        return carry
    lax.fori_loop(0, NCHUNK, newton_chunk, 0)

    dense_stage(y1h, 16, buf16, 16, buf16,
                [("sp", acc16), ("hbmflat", tbl16h)], "copy", src_is_hbm=True)
    plsc.subcore_barrier()

    edge_pass(tbl16h, acc16, buf16)
    plsc.subcore_barrier()
    load_wb(OFF_D1, 2048, BOFF_D1, 256)
    dense_stage(acc16, 16, buf16, 8, buf8,
                [("hbmflat", tbl8h), ("sp", acc8)], "pre")
    plsc.subcore_barrier()

    edge_pass(tbl8h, acc8, buf8)
    plsc.subcore_barrier()
    load_wb(OFF_D2, 512, BOFF_D2, 128)
    dense_stage(acc8, 8, buf8, 4, buf8,
                [("hbmflat", tbl8h), ("sp", acc8)], "pre", w_pad=8)
    plsc.subcore_barrier()

    def stage_body(s, carry):
        edge_pass(tbl8h, acc8, buf8)
        plsc.subcore_barrier()
        load_wb(OFF_DLOOP + s * 256, 256, BOFF_DLOOP + s * 64, 64)
        dense_stage(acc8, 4, buf8, 4, buf8,
                    [("hbmflat", tbl8h), ("sp", acc8)], "pre", w_pad=8)
        plsc.subcore_barrier()
        return carry
    lax.fori_loop(0, 21, stage_body, 0)

    edge_pass(tbl8h, acc8, buf8)
    plsc.subcore_barrier()
    load_wb(OFF_D24, 512, BOFF_D24, 128)
    dense_stage(acc8, 4, buf8, 8, buf8,
                [("hbmflat", tbl8h), ("sp", acc8)], "post")
    plsc.subcore_barrier()

    edge_pass(tbl8h, acc8, buf8)
    plsc.subcore_barrier()
    load_wb(OFF_D25, 2048, BOFF_D25, 256)
    dense_stage(acc8, 8, buf8, 16, buf16,
                [("sp", acc16), ("hbmflat", tbl16h)], "post")
    plsc.subcore_barrier()

    edge_pass(tbl16h, acc16, buf16)
    plsc.subcore_barrier()
    dense_stage(acc16, 16, buf16, 16, buf16, [("hbm2", u26h)], "copy")


def _sc_call(y1, srcah, dsth, wvh, bvh):
    mesh = plsc.VectorSubcoreMesh(core_axis_name="c", subcore_axis_name="s",
                                  num_cores=2, num_subcores=NS)
    f = functools.partial(
        pl.kernel,
        out_type=(jax.ShapeDtypeStruct((2, N, 16), jnp.float32),
                  jax.ShapeDtypeStruct((2 * N, 8), jnp.float32),
                  jax.ShapeDtypeStruct((2 * N, 16), jnp.float32)),
        mesh=mesh,
        compiler_params=pltpu.CompilerParams(
            needs_layout_passes=False, use_tc_tiling_on_sc=False),
        scratch_types=[
            pltpu.VMEM_SHARED((N, 8), jnp.float32),
            pltpu.VMEM_SHARED((N, 16), jnp.float32),
            pltpu.VMEM_SHARED((N, 8), jnp.float32),
            pltpu.VMEM((KF, BLK), jnp.int32),
            pltpu.VMEM((KF, BLK), jnp.int32),
            pltpu.VMEM((KF * BLK, 8), jnp.float32),
            pltpu.VMEM((KF * BLK, 16), jnp.float32),
            pltpu.VMEM((CHUNK, 8), jnp.float32),
            pltpu.VMEM((NODES_T,), jnp.float32),
            pltpu.VMEM((2048,), jnp.float32),
            pltpu.VMEM((256,), jnp.float32),
            pltpu.SemaphoreType.DMA,
            pltpu.SemaphoreType.DMA,
        ],
    )(_sc_body)
    return f(y1, srcah, dsth, wvh, bvh)


def _mm_in_kernel(x_ref, w_ref, o_ref):
    o_ref[0] = jnp.dot(x_ref[0], w_ref[0], preferred_element_type=jnp.float32)


def _mm_in(x, w):
    bn = 8000
    return pl.pallas_call(
        _mm_in_kernel,
        grid=(2, N // bn),
        in_specs=[
            pl.BlockSpec((1, bn, 128), lambda m, i: (m, i, 0)),
            pl.BlockSpec((1, 128, 16), lambda m, i: (m, 0, 0)),
        ],
        out_specs=pl.BlockSpec((1, bn, 16), lambda m, i: (m, i, 0)),
        out_shape=jax.ShapeDtypeStruct((2, N, 16), jnp.float32),
    )(x, w)


def _mm_out_kernel(u_ref, w_ref, b_ref, o_ref):
    o_ref[0] = (jnp.dot(u_ref[0], w_ref[0], preferred_element_type=jnp.float32)
                + b_ref[0])


def _mm_out(u, w, b):
    bn = 8000
    return pl.pallas_call(
        _mm_out_kernel,
        grid=(2, N // bn),
        in_specs=[
            pl.BlockSpec((1, bn, 16), lambda m, i: (m, i, 0)),
            pl.BlockSpec((1, 16, 128), lambda m, i: (m, 0, 0)),
            pl.BlockSpec((1, 1, 128), lambda m, i: (m, 0, 0)),
        ],
        out_specs=pl.BlockSpec((1, bn, 128), lambda m, i: (m, i, 0)),
        out_shape=jax.ShapeDtypeStruct((2, N, 128), jnp.float32),
    )(u, w, b)


def _bc_flat(a):
    return jnp.repeat(a.reshape(-1)[:, None], 16, axis=1).reshape(-1)


def _build_tables(params, i):
    wpieces = [_bc_flat(params[f"W_{i}_2"]), _bc_flat(params[f"W_{i}_3"])]
    for s in range(3, 23):
        wpieces.append(_bc_flat(params[f"W_{i}_{s + 1}"]))
    wpieces.append(_bc_flat(jnp.eye(4, dtype=jnp.float32)))
    wpieces.append(_bc_flat(params[f"W_{i}_24"]))
    wpieces.append(_bc_flat(params[f"W_{i}_25"]))
    bpieces = [_bc_flat(params[f"b_{i}_{j}"]) for j in range(1, 26)]
    return jnp.concatenate(wpieces), jnp.concatenate(bpieces)


def kernel(inp_0, edge_index_0, inp_1, edge_index_1, params):
    x0 = jnp.stack([inp_0, inp_1])
    w1 = jnp.stack([params["W_0_1"], params["W_1_1"]])
    y1 = _mm_in(x0, w1)

    srch = jnp.stack([edge_index_0[0], edge_index_1[0]]).reshape(
        2, NS, NSB, KF, BLK)
    srcah = srch + (jnp.arange(2, dtype=jnp.int32) * N).reshape(2, 1, 1, 1, 1)
    dsth = jnp.stack([edge_index_0[1], edge_index_1[1]]).reshape(
        2, NS, NSB, KF, BLK)
    wv0, bv0 = _build_tables(params, 0)
    wv1, bv1 = _build_tables(params, 1)
    wvh = jnp.concatenate([wv0, wv1])
    bvh = jnp.concatenate([bv0, bv1])

    u26 = _sc_call(y1, srcah, dsth, wvh, bvh)[0]

    w26 = jnp.stack([params["W_0_26"], params["W_1_26"]])
    b26 = jnp.stack([params["b_0_26"], params["b_1_26"]])[:, None, :]
    outs = _mm_out(u26, w26, b26)
    return (outs[0], outs[1])


# speedup vs baseline: 70.3297x; 1.8401x over previous
import functools

import jax
import jax.numpy as jnp
from jax import lax
from jax.experimental import pallas as pl
from jax.experimental.pallas import tpu as pltpu
from jax.experimental.pallas import tpu_sc as plsc

N = 32000
E = 512000
NS = 16
NODES_T = N // NS
EDGES_T = E // NS
BLK = 128
KF = 10
NSB = EDGES_T // (BLK * KF)
CHUNK = 400
NCHUNK = NODES_T // CHUNK
NGRP = CHUNK // 16

WV_LEN = 10496
BV_LEN = 2112
OFF_D1, OFF_D2, OFF_DLOOP, OFF_D24, OFF_D25 = 0, 2048, 2560, 7936, 8448
BOFF_D1, BOFF_D2, BOFF_DLOOP, BOFF_D24, BOFF_D25 = 0, 256, 384, 1728, 1856


def _sc_body(y1h, srcah, srch, dsth, wvh, bvh, u26h, tbl16h,
             acc8, acc16, table8,
             sidxb, didxb, buf8, buf16, ones2, dinv_t, wvs, bvs,
             gsem, ssem):
    cid = lax.axis_index("c")
    tid = lax.axis_index("s")
    iota = lax.iota(jnp.int32, 16)

    def edge_pass(tbl, acc, buf, src_hbm, kf):
        nsb = 250 // kf

        def sb_body(sb, carry):
            pltpu.sync_copy(src_hbm.at[cid, tid, pl.ds(sb * kf, kf), :],
                            sidxb.at[pl.ds(0, kf), :])
            pltpu.sync_copy(dsth.at[cid, tid, pl.ds(sb * kf, kf), :],
                            didxb.at[pl.ds(0, kf), :])
            gds = []
            for j in range(kf):
                gds.append(pltpu.async_copy(
                    tbl.at[sidxb.at[j]], buf.at[pl.ds(j * BLK, BLK), :],
                    gsem))
            sds = []
            for j in range(kf):
                gds[j].wait()
                sds.append(pltpu.async_copy(
                    buf.at[pl.ds(j * BLK, BLK), :], acc.at[didxb.at[j]],
                    ssem, add=True))
            for d in sds:
                d.wait()
            return carry
        lax.fori_loop(0, nsb, sb_body, 0)

    def load_wb(wv_off, wv_len, bv_off, bv_len):
        pltpu.sync_copy(
            wvh.at[pl.ds(pl.multiple_of(cid * WV_LEN + wv_off, 16), wv_len)],
            wvs.at[pl.ds(0, wv_len)])
        pltpu.sync_copy(
            bvh.at[pl.ds(pl.multiple_of(cid * BV_LEN + bv_off, 16), bv_len)],
            bvs.at[pl.ds(0, bv_len)])

    def dense_stage(src_ref, w_in, buf_in, w_out, buf_out, outs, mode,
                    src_is_hbm=False, w_pad=0):
        def chunk_body(c, carry):
            nbase = pl.multiple_of(tid * NODES_T + c * CHUNK, 16)
            if src_is_hbm:
                pltpu.sync_copy(src_ref.at[cid, pl.ds(nbase, CHUNK), :],
                                buf_in.at[pl.ds(0, CHUNK), :])
            else:
                pltpu.sync_copy(src_ref.at[pl.ds(nbase, CHUNK), :],
                                buf_in.at[pl.ds(0, CHUNK), :])

            def grp(j, carry2):
                rows = j * 16 + iota
                dv = dinv_t[pl.ds(pl.multiple_of(c * CHUNK + j * 16, 16), 16)]
                xs = []
                for ci in range(w_in):
                    v = plsc.load_gather(
                        buf_in, [rows, jnp.full((16,), ci, jnp.int32)])
                    v = v * dv
                    if mode == "pre":
                        b = bvs[pl.ds(ci * 16, 16)]
                        v = jnp.maximum(v + b, 0.0)
                    xs.append(v)
                if mode == "copy":
                    for co in range(w_out):
                        plsc.store_scatter(
                            buf_out, [rows, jnp.full((16,), co, jnp.int32)],
                            xs[co])
                else:
                    for co in range(w_out):
                        accv = None
                        for ci in range(w_in):
                            wvec = wvs[pl.ds((ci * w_out + co) * 16, 16)]
                            t = xs[ci] * wvec
                            accv = t if accv is None else accv + t
                        if mode == "post":
                            b = bvs[pl.ds(co * 16, 16)]
                            accv = jnp.maximum(accv + b, 0.0)
                        accv = accv * dv
                        plsc.store_scatter(
                            buf_out, [rows, jnp.full((16,), co, jnp.int32)],
                            accv)
                    for co in range(w_out, w_pad):
                        plsc.store_scatter(
                            buf_out, [rows, jnp.full((16,), co, jnp.int32)],
                            jnp.full((16,), 0.0, jnp.float32))
                return carry2
            lax.fori_loop(0, NGRP, grp, 0)

            src_chunk = buf_out.at[pl.ds(0, CHUNK), :]
            for kind, ref in outs:
                if kind == "sp":
                    pltpu.sync_copy(src_chunk, ref.at[pl.ds(nbase, CHUNK), :])
                elif kind == "hbm2":
                    pltpu.sync_copy(src_chunk,
                                    ref.at[cid, pl.ds(nbase, CHUNK), :])
                else:
                    pltpu.sync_copy(
                        src_chunk,
                        ref.at[pl.ds(pl.multiple_of(cid * N + nbase, 16), CHUNK), :])
            return carry
        lax.fori_loop(0, NCHUNK, chunk_body, 0)

    def fill_ones(g, carry):
        f = g * 16 + iota
        plsc.store_scatter(ones2, [f >> 3, f & 7],
                           jnp.full((16,), 1.0, jnp.float32))
        return carry
    lax.fori_loop(0, 400 * 8 // 16, fill_ones, 0)

    def deg_init(c, carry):
        pltpu.sync_copy(
            ones2,
            acc8.at[pl.ds(pl.multiple_of(tid * NODES_T + c * CHUNK, 16),
                          CHUNK), :])
        return carry
    lax.fori_loop(0, NCHUNK, deg_init, 0)
    plsc.subcore_barrier()

    def deg_sb(sb, carry):
        pltpu.sync_copy(dsth.at[cid, tid, pl.ds(sb * 25, 25), :], didxb)
        descs = []
        for j in range(25):
            descs.append(pltpu.async_copy(
                ones2.at[pl.ds(0, BLK), :], acc8.at[didxb.at[j]],
                ssem, add=True))
        for d in descs:
            d.wait()
        return carry
    lax.fori_loop(0, 10, deg_sb, 0)
    plsc.subcore_barrier()

    def newton_chunk(c, carry):
        pltpu.sync_copy(
            acc8.at[pl.ds(pl.multiple_of(tid * NODES_T + c * CHUNK, 16),
                          CHUNK), :],
            buf8.at[pl.ds(0, CHUNK), :])

        def newton(j, carry2):
            rows = j * 16 + iota
            v = plsc.load_gather(buf8, [rows, jnp.full((16,), 0, jnp.int32)])
            bits = lax.bitcast_convert_type(v, jnp.int32)
            y = lax.bitcast_convert_type(
                jnp.int32(0x5F3759DF) - (bits >> 1), jnp.float32)
            for _ in range(3):
                y = y * (1.5 - 0.5 * v * y * y)
            dinv_t[pl.ds(pl.multiple_of(c * CHUNK + j * 16, 16), 16)] = y
            return carry2
        lax.fori_loop(0, NGRP, newton, 0)
        return carry
    lax.fori_loop(0, NCHUNK, newton_chunk, 0)

    dense_stage(y1h, 16, buf16, 16, buf16,
                [("sp", acc16), ("hbmflat", tbl16h)], "copy", src_is_hbm=True)
    plsc.subcore_barrier()

    edge_pass(tbl16h, acc16, buf16, srcah, KF)
    plsc.subcore_barrier()
    load_wb(OFF_D1, 2048, BOFF_D1, 256)
    dense_stage(acc16, 16, buf16, 8, buf8,
                [("sp", table8), ("sp", acc8)], "pre")
    plsc.subcore_barrier()

    edge_pass(table8, acc8, buf8, srch, 25)
    plsc.subcore_barrier()
    load_wb(OFF_D2, 512, BOFF_D2, 128)
    dense_stage(acc8, 8, buf8, 4, buf8,
                [("sp", table8), ("sp", acc8)], "pre", w_pad=8)
    plsc.subcore_barrier()

    def stage_body(s, carry):
        edge_pass(table8, acc8, buf8, srch, 25)
        plsc.subcore_barrier()
        load_wb(OFF_DLOOP + s * 256, 256, BOFF_DLOOP + s * 64, 64)
        dense_stage(acc8, 4, buf8, 4, buf8,
                    [("sp", table8), ("sp", acc8)], "pre", w_pad=8)
        plsc.subcore_barrier()
        return carry
    lax.fori_loop(0, 21, stage_body, 0)

    edge_pass(table8, acc8, buf8, srch, 25)
    plsc.subcore_barrier()
    load_wb(OFF_D24, 512, BOFF_D24, 128)
    dense_stage(acc8, 4, buf8, 8, buf8,
                [("sp", table8), ("sp", acc8)], "post")
    plsc.subcore_barrier()

    edge_pass(table8, acc8, buf8, srch, 25)
    plsc.subcore_barrier()
    load_wb(OFF_D25, 2048, BOFF_D25, 256)
    dense_stage(acc8, 8, buf8, 16, buf16,
                [("sp", acc16), ("hbmflat", tbl16h)], "post")
    plsc.subcore_barrier()

    edge_pass(tbl16h, acc16, buf16, srcah, KF)
    plsc.subcore_barrier()
    dense_stage(acc16, 16, buf16, 16, buf16, [("hbm2", u26h)], "copy")


def _sc_call(y1, srcah, srch, dsth, wvh, bvh):
    mesh = plsc.VectorSubcoreMesh(core_axis_name="c", subcore_axis_name="s",
                                  num_cores=2, num_subcores=NS)
    f = functools.partial(
        pl.kernel,
        out_type=(jax.ShapeDtypeStruct((2, N, 16), jnp.float32),
                  jax.ShapeDtypeStruct((2 * N, 16), jnp.float32)),
        mesh=mesh,
        compiler_params=pltpu.CompilerParams(
            needs_layout_passes=False, use_tc_tiling_on_sc=False),
        scratch_types=[
            pltpu.VMEM_SHARED((N, 8), jnp.float32),
            pltpu.VMEM_SHARED((N, 16), jnp.float32),
            pltpu.VMEM_SHARED((N, 8), jnp.float32),
            pltpu.VMEM((25, BLK), jnp.int32),
            pltpu.VMEM((25, BLK), jnp.int32),
            pltpu.VMEM((25 * BLK, 8), jnp.float32),
            pltpu.VMEM((KF * BLK, 16), jnp.float32),
            pltpu.VMEM((CHUNK, 8), jnp.float32),
            pltpu.VMEM((NODES_T,), jnp.float32),
            pltpu.VMEM((2048,), jnp.float32),
            pltpu.VMEM((256,), jnp.float32),
            pltpu.SemaphoreType.DMA,
            pltpu.SemaphoreType.DMA,
        ],
    )(_sc_body)
    return f(y1, srcah, srch, dsth, wvh, bvh)


def _mm_in_kernel(x_ref, w_ref, o_ref):
    o_ref[0] = jnp.dot(x_ref[0], w_ref[0], preferred_element_type=jnp.float32)


def _mm_in(x, w):
    bn = 8000
    return pl.pallas_call(
        _mm_in_kernel,
        grid=(2, N // bn),
        in_specs=[
            pl.BlockSpec((1, bn, 128), lambda m, i: (m, i, 0)),
            pl.BlockSpec((1, 128, 16), lambda m, i: (m, 0, 0)),
        ],
        out_specs=pl.BlockSpec((1, bn, 16), lambda m, i: (m, i, 0)),
        out_shape=jax.ShapeDtypeStruct((2, N, 16), jnp.float32),
    )(x, w)


def _mm_out_kernel(u_ref, w_ref, b_ref, o_ref):
    o_ref[0] = (jnp.dot(u_ref[0], w_ref[0], preferred_element_type=jnp.float32)
                + b_ref[0])


def _mm_out(u, w, b):
    bn = 8000
    return pl.pallas_call(
        _mm_out_kernel,
        grid=(2, N // bn),
        in_specs=[
            pl.BlockSpec((1, bn, 16), lambda m, i: (m, i, 0)),
            pl.BlockSpec((1, 16, 128), lambda m, i: (m, 0, 0)),
            pl.BlockSpec((1, 1, 128), lambda m, i: (m, 0, 0)),
        ],
        out_specs=pl.BlockSpec((1, bn, 128), lambda m, i: (m, i, 0)),
        out_shape=jax.ShapeDtypeStruct((2, N, 128), jnp.float32),
    )(u, w, b)


def _bc_flat(a):
    return jnp.repeat(a.reshape(-1)[:, None], 16, axis=1).reshape(-1)


def _build_tables(params, i):
    wpieces = [_bc_flat(params[f"W_{i}_2"]), _bc_flat(params[f"W_{i}_3"])]
    for s in range(3, 23):
        wpieces.append(_bc_flat(params[f"W_{i}_{s + 1}"]))
    wpieces.append(_bc_flat(jnp.eye(4, dtype=jnp.float32)))
    wpieces.append(_bc_flat(params[f"W_{i}_24"]))
    wpieces.append(_bc_flat(params[f"W_{i}_25"]))
    bpieces = [_bc_flat(params[f"b_{i}_{j}"]) for j in range(1, 26)]
    return jnp.concatenate(wpieces), jnp.concatenate(bpieces)


def kernel(inp_0, edge_index_0, inp_1, edge_index_1, params):
    x0 = jnp.stack([inp_0, inp_1])
    w1 = jnp.stack([params["W_0_1"], params["W_1_1"]])
    y1 = _mm_in(x0, w1)

    srch = jnp.stack([edge_index_0[0], edge_index_1[0]]).reshape(
        2, NS, 250, BLK)
    srcah = srch + (jnp.arange(2, dtype=jnp.int32) * N).reshape(2, 1, 1, 1)
    dsth = jnp.stack([edge_index_0[1], edge_index_1[1]]).reshape(
        2, NS, 250, BLK)
    wv0, bv0 = _build_tables(params, 0)
    wv1, bv1 = _build_tables(params, 1)
    wvh = jnp.concatenate([wv0, wv1])
    bvh = jnp.concatenate([bv0, bv1])

    u26 = _sc_call(y1, srcah, srch, dsth, wvh, bvh)[0]

    w26 = jnp.stack([params["W_0_26"], params["W_1_26"]])
    b26 = jnp.stack([params["b_0_26"], params["b_1_26"]])[:, None, :]
    outs = _mm_out(u26, w26, b26)
    return (outs[0], outs[1])


# speedup vs baseline: 76.6096x; 1.0893x over previous
import functools

import jax
import jax.numpy as jnp
from jax import lax
from jax.experimental import pallas as pl
from jax.experimental.pallas import tpu as pltpu
from jax.experimental.pallas import tpu_sc as plsc

N = 32000
E = 512000
NS = 16
NODES_T = N // NS
EDGES_T = E // NS
BLK = 128
KF = 10
NSB = EDGES_T // (BLK * KF)
CHUNK = 400
NCHUNK = NODES_T // CHUNK
NGRP = CHUNK // 16

WV_LEN = 10496
BV_LEN = 2112
OFF_D1, OFF_D2, OFF_DLOOP, OFF_D24, OFF_D25 = 0, 2048, 2560, 7936, 8448
BOFF_D1, BOFF_D2, BOFF_DLOOP, BOFF_D24, BOFF_D25 = 0, 256, 384, 1728, 1856


def _sc_body(y1h, srcah, srch, dsth, wvh, bvh, u26h, tbl16h,
             acc8, acc16, table8,
             sidxb, didxb, buf8, buf16, ones2, dinv_t, wvs, bvs,
             gsem, ssem, isem):
    cid = lax.axis_index("c")
    tid = lax.axis_index("s")
    iota = lax.iota(jnp.int32, 16)

    def edge_pass(tbl, acc, buf, src_hbm, kf):
        nsb = 250 // kf

        def sb_body(sb, carry):
            pltpu.sync_copy(src_hbm.at[cid, tid, pl.ds(sb * kf, kf), :],
                            sidxb.at[pl.ds(0, kf), :])
            pltpu.sync_copy(dsth.at[cid, tid, pl.ds(sb * kf, kf), :],
                            didxb.at[pl.ds(0, kf), :])
            gds = []
            for j in range(kf):
                gds.append(pltpu.async_copy(
                    tbl.at[sidxb.at[j]], buf.at[pl.ds(j * BLK, BLK), :],
                    gsem))
            sds = []
            for j in range(kf):
                gds[j].wait()
                sds.append(pltpu.async_copy(
                    buf.at[pl.ds(j * BLK, BLK), :], acc.at[didxb.at[j]],
                    ssem, add=True))
            for d in sds:
                d.wait()
            return carry
        lax.fori_loop(0, nsb, sb_body, 0)

    def edge_pass_db(tbl, acc, buf, src_hbm, kf):
        nsb = 250 // kf

        def idx_load(rnd, half):
            off = jnp.minimum(rnd, nsb - 1) * kf
            a = pltpu.async_copy(
                src_hbm.at[cid, tid, pl.ds(off, kf), :],
                sidxb.at[pl.ds(half * kf, kf), :], isem)
            b = pltpu.async_copy(
                dsth.at[cid, tid, pl.ds(off, kf), :],
                didxb.at[pl.ds(half * kf, kf), :], isem)
            return a, b

        def idx_wait(half):
            for ref, dst in ((src_hbm, sidxb), (dsth, didxb)):
                pltpu.make_async_copy(
                    ref.at[cid, tid, pl.ds(0, kf), :],
                    dst.at[pl.ds(half * kf, kf), :], isem).wait()

        def half_round(half, acc_, buf_, tbl_):
            gds = []
            for j in range(kf):
                gds.append(pltpu.async_copy(
                    tbl_.at[sidxb.at[half * kf + j]],
                    buf_.at[pl.ds(j * BLK, BLK), :], gsem))
            return gds

        idx_load(0, 0)

        def pair_body(r, carry):
            for half in (0, 1):
                rnd = 2 * r + half
                idx_wait(half)
                gds = half_round(half, acc, buf, tbl)
                idx_load(rnd + 1, 1 - half)
                sds = []
                for j in range(kf):
                    gds[j].wait()
                    sds.append(pltpu.async_copy(
                        buf.at[pl.ds(j * BLK, BLK), :],
                        acc.at[didxb.at[half * kf + j]], ssem, add=True))
                for d in sds:
                    d.wait()
            return carry
        lax.fori_loop(0, nsb // 2, pair_body, 0)
        idx_wait(nsb % 2)

    def load_wb(wv_off, wv_len, bv_off, bv_len):
        pltpu.sync_copy(
            wvh.at[pl.ds(pl.multiple_of(cid * WV_LEN + wv_off, 16), wv_len)],
            wvs.at[pl.ds(0, wv_len)])
        pltpu.sync_copy(
            bvh.at[pl.ds(pl.multiple_of(cid * BV_LEN + bv_off, 16), bv_len)],
            bvs.at[pl.ds(0, bv_len)])

    def dense_stage(src_ref, w_in, buf_in, w_out, buf_out, outs, mode,
                    src_is_hbm=False, w_pad=0):
        def chunk_body(c, carry):
            nbase = pl.multiple_of(tid * NODES_T + c * CHUNK, 16)
            if src_is_hbm:
                pltpu.sync_copy(src_ref.at[cid, pl.ds(nbase, CHUNK), :],
                                buf_in.at[pl.ds(0, CHUNK), :])
            else:
                pltpu.sync_copy(src_ref.at[pl.ds(nbase, CHUNK), :],
                                buf_in.at[pl.ds(0, CHUNK), :])

            def grp(j, carry2):
                rows = j * 16 + iota
                dv = dinv_t[pl.ds(pl.multiple_of(c * CHUNK + j * 16, 16), 16)]
                xs = []
                for ci in range(w_in):
                    v = plsc.load_gather(
                        buf_in, [rows, jnp.full((16,), ci, jnp.int32)])
                    v = v * dv
                    if mode == "pre":
                        b = bvs[pl.ds(ci * 16, 16)]
                        v = jnp.maximum(v + b, 0.0)
                    xs.append(v)
                if mode == "copy":
                    for co in range(w_out):
                        plsc.store_scatter(
                            buf_out, [rows, jnp.full((16,), co, jnp.int32)],
                            xs[co])
                else:
                    for co in range(w_out):
                        accv = None
                        for ci in range(w_in):
                            wvec = wvs[pl.ds((ci * w_out + co) * 16, 16)]
                            t = xs[ci] * wvec
                            accv = t if accv is None else accv + t
                        if mode == "post":
                            b = bvs[pl.ds(co * 16, 16)]
                            accv = jnp.maximum(accv + b, 0.0)
                        accv = accv * dv
                        plsc.store_scatter(
                            buf_out, [rows, jnp.full((16,), co, jnp.int32)],
                            accv)
                    for co in range(w_out, w_pad):
                        plsc.store_scatter(
                            buf_out, [rows, jnp.full((16,), co, jnp.int32)],
                            jnp.full((16,), 0.0, jnp.float32))
                return carry2
            lax.fori_loop(0, NGRP, grp, 0)

            src_chunk = buf_out.at[pl.ds(0, CHUNK), :]
            for kind, ref in outs:
                if kind == "sp":
                    pltpu.sync_copy(src_chunk, ref.at[pl.ds(nbase, CHUNK), :])
                elif kind == "hbm2":
                    pltpu.sync_copy(src_chunk,
                                    ref.at[cid, pl.ds(nbase, CHUNK), :])
                else:
                    pltpu.sync_copy(
                        src_chunk,
                        ref.at[pl.ds(pl.multiple_of(cid * N + nbase, 16), CHUNK), :])
            return carry
        lax.fori_loop(0, NCHUNK, chunk_body, 0)

    def fill_ones(g, carry):
        f = g * 16 + iota
        plsc.store_scatter(ones2, [f >> 3, f & 7],
                           jnp.full((16,), 1.0, jnp.float32))
        return carry
    lax.fori_loop(0, 400 * 8 // 16, fill_ones, 0)

    def deg_init(c, carry):
        pltpu.sync_copy(
            ones2,
            acc8.at[pl.ds(pl.multiple_of(tid * NODES_T + c * CHUNK, 16),
                          CHUNK), :])
        return carry
    lax.fori_loop(0, NCHUNK, deg_init, 0)
    plsc.subcore_barrier()

    def deg_sb(sb, carry):
        pltpu.sync_copy(dsth.at[cid, tid, pl.ds(sb * 25, 25), :],
                        didxb.at[pl.ds(0, 25), :])
        descs = []
        for j in range(25):
            descs.append(pltpu.async_copy(
                ones2.at[pl.ds(0, BLK), :], acc8.at[didxb.at[j]],
                ssem, add=True))
        for d in descs:
            d.wait()
        return carry
    lax.fori_loop(0, 10, deg_sb, 0)
    plsc.subcore_barrier()

    def newton_chunk(c, carry):
        pltpu.sync_copy(
            acc8.at[pl.ds(pl.multiple_of(tid * NODES_T + c * CHUNK, 16),
                          CHUNK), :],
            buf8.at[pl.ds(0, CHUNK), :])

        def newton(j, carry2):
            rows = j * 16 + iota
            v = plsc.load_gather(buf8, [rows, jnp.full((16,), 0, jnp.int32)])
            bits = lax.bitcast_convert_type(v, jnp.int32)
            y = lax.bitcast_convert_type(
                jnp.int32(0x5F3759DF) - (bits >> 1), jnp.float32)
            for _ in range(3):
                y = y * (1.5 - 0.5 * v * y * y)
            dinv_t[pl.ds(pl.multiple_of(c * CHUNK + j * 16, 16), 16)] = y
            return carry2
        lax.fori_loop(0, NGRP, newton, 0)
        return carry
    lax.fori_loop(0, NCHUNK, newton_chunk, 0)

    dense_stage(y1h, 16, buf16, 16, buf16,
                [("sp", acc16), ("hbmflat", tbl16h)], "copy", src_is_hbm=True)
    plsc.subcore_barrier()

    edge_pass(tbl16h, acc16, buf16, srcah, KF)
    plsc.subcore_barrier()
    load_wb(OFF_D1, 2048, BOFF_D1, 256)
    dense_stage(acc16, 16, buf16, 8, buf8,
                [("sp", table8), ("sp", acc8)], "pre")
    plsc.subcore_barrier()

    edge_pass_db(table8, acc8, buf8, srch, 25)
    plsc.subcore_barrier()
    load_wb(OFF_D2, 512, BOFF_D2, 128)
    dense_stage(acc8, 8, buf8, 4, buf8,
                [("sp", table8), ("sp", acc8)], "pre", w_pad=8)
    plsc.subcore_barrier()

    def stage_body(s, carry):
        edge_pass_db(table8, acc8, buf8, srch, 25)
        plsc.subcore_barrier()
        load_wb(OFF_DLOOP + s * 256, 256, BOFF_DLOOP + s * 64, 64)
        dense_stage(acc8, 4, buf8, 4, buf8,
                    [("sp", table8), ("sp", acc8)], "pre", w_pad=8)
        plsc.subcore_barrier()
        return carry
    lax.fori_loop(0, 21, stage_body, 0)

    edge_pass_db(table8, acc8, buf8, srch, 25)
    plsc.subcore_barrier()
    load_wb(OFF_D24, 512, BOFF_D24, 128)
    dense_stage(acc8, 4, buf8, 8, buf8,
                [("sp", table8), ("sp", acc8)], "post")
    plsc.subcore_barrier()

    edge_pass_db(table8, acc8, buf8, srch, 25)
    plsc.subcore_barrier()
    load_wb(OFF_D25, 2048, BOFF_D25, 256)
    dense_stage(acc8, 8, buf8, 16, buf16,
                [("sp", acc16), ("hbmflat", tbl16h)], "post")
    plsc.subcore_barrier()

    edge_pass(tbl16h, acc16, buf16, srcah, KF)
    plsc.subcore_barrier()
    dense_stage(acc16, 16, buf16, 16, buf16, [("hbm2", u26h)], "copy")


def _sc_call(y1, srcah, srch, dsth, wvh, bvh):
    mesh = plsc.VectorSubcoreMesh(core_axis_name="c", subcore_axis_name="s",
                                  num_cores=2, num_subcores=NS)
    f = functools.partial(
        pl.kernel,
        out_type=(jax.ShapeDtypeStruct((2, N, 16), jnp.float32),
                  jax.ShapeDtypeStruct((2 * N, 16), jnp.float32)),
        mesh=mesh,
        compiler_params=pltpu.CompilerParams(
            needs_layout_passes=False, use_tc_tiling_on_sc=False),
        scratch_types=[
            pltpu.VMEM_SHARED((N, 8), jnp.float32),
            pltpu.VMEM_SHARED((N, 16), jnp.float32),
            pltpu.VMEM_SHARED((N, 8), jnp.float32),
            pltpu.VMEM((50, BLK), jnp.int32),
            pltpu.VMEM((50, BLK), jnp.int32),
            pltpu.VMEM((25 * BLK, 8), jnp.float32),
            pltpu.VMEM((KF * BLK, 16), jnp.float32),
            pltpu.VMEM((CHUNK, 8), jnp.float32),
            pltpu.VMEM((NODES_T,), jnp.float32),
            pltpu.VMEM((2048,), jnp.float32),
            pltpu.VMEM((256,), jnp.float32),
            pltpu.SemaphoreType.DMA,
            pltpu.SemaphoreType.DMA,
            pltpu.SemaphoreType.DMA,
        ],
    )(_sc_body)
    return f(y1, srcah, srch, dsth, wvh, bvh)


def _mm_in_kernel(x_ref, w_ref, o_ref):
    o_ref[0] = jnp.dot(x_ref[0], w_ref[0], preferred_element_type=jnp.float32)


def _mm_in(x, w):
    bn = 8000
    return pl.pallas_call(
        _mm_in_kernel,
        grid=(2, N // bn),
        in_specs=[
            pl.BlockSpec((1, bn, 128), lambda m, i: (m, i, 0)),
            pl.BlockSpec((1, 128, 16), lambda m, i: (m, 0, 0)),
        ],
        out_specs=pl.BlockSpec((1, bn, 16), lambda m, i: (m, i, 0)),
        out_shape=jax.ShapeDtypeStruct((2, N, 16), jnp.float32),
    )(x, w)


def _mm_out_kernel(u_ref, w_ref, b_ref, o_ref):
    o_ref[0] = (jnp.dot(u_ref[0], w_ref[0], preferred_element_type=jnp.float32)
                + b_ref[0])


def _mm_out(u, w, b):
    bn = 8000
    return pl.pallas_call(
        _mm_out_kernel,
        grid=(2, N // bn),
        in_specs=[
            pl.BlockSpec((1, bn, 16), lambda m, i: (m, i, 0)),
            pl.BlockSpec((1, 16, 128), lambda m, i: (m, 0, 0)),
            pl.BlockSpec((1, 1, 128), lambda m, i: (m, 0, 0)),
        ],
        out_specs=pl.BlockSpec((1, bn, 128), lambda m, i: (m, i, 0)),
        out_shape=jax.ShapeDtypeStruct((2, N, 128), jnp.float32),
    )(u, w, b)


def _bc_flat(a):
    return jnp.repeat(a.reshape(-1)[:, None], 16, axis=1).reshape(-1)


def _build_tables(params, i):
    wpieces = [_bc_flat(params[f"W_{i}_2"]), _bc_flat(params[f"W_{i}_3"])]
    for s in range(3, 23):
        wpieces.append(_bc_flat(params[f"W_{i}_{s + 1}"]))
    wpieces.append(_bc_flat(jnp.eye(4, dtype=jnp.float32)))
    wpieces.append(_bc_flat(params[f"W_{i}_24"]))
    wpieces.append(_bc_flat(params[f"W_{i}_25"]))
    bpieces = [_bc_flat(params[f"b_{i}_{j}"]) for j in range(1, 26)]
    return jnp.concatenate(wpieces), jnp.concatenate(bpieces)


def kernel(inp_0, edge_index_0, inp_1, edge_index_1, params):
    x0 = jnp.stack([inp_0, inp_1])
    w1 = jnp.stack([params["W_0_1"], params["W_1_1"]])
    y1 = _mm_in(x0, w1)

    srch = jnp.stack([edge_index_0[0], edge_index_1[0]]).reshape(
        2, NS, 250, BLK)
    srcah = srch + (jnp.arange(2, dtype=jnp.int32) * N).reshape(2, 1, 1, 1)
    dsth = jnp.stack([edge_index_0[1], edge_index_1[1]]).reshape(
        2, NS, 250, BLK)
    wv0, bv0 = _build_tables(params, 0)
    wv1, bv1 = _build_tables(params, 1)
    wvh = jnp.concatenate([wv0, wv1])
    bvh = jnp.concatenate([bv0, bv1])

    u26 = _sc_call(y1, srcah, srch, dsth, wvh, bvh)[0]

    w26 = jnp.stack([params["W_0_26"], params["W_1_26"]])
    b26 = jnp.stack([params["b_0_26"], params["b_1_26"]])[:, None, :]
    outs = _mm_out(u26, w26, b26)
    return (outs[0], outs[1])
